# baseline (speedup 1.0000x reference)
import functools

import jax
import jax.numpy as jnp
from jax import lax
from jax.experimental import pallas as pl
from jax.experimental.pallas import tpu as pltpu
from jax.experimental.pallas import tpu_sc as plsc

EMBED = 64
LANES = 16
NUM_WORKERS = 32
CHUNK = 800
SCALE = 8.0


def _body(x_hbm, tab_hbm, out_hbm, idx_v, rows_v, sem):
    wid = lax.axis_index("s") * 2 + lax.axis_index("c")
    n_total = x_hbm.shape[0]
    per_w = n_total // NUM_WORKERS
    n_chunks = per_w // CHUNK
    base = wid * per_w

    def chunk_body(c, carry):
        start = base + c * CHUNK
        pltpu.sync_copy(x_hbm.at[pl.ds(start, CHUNK)], idx_v)
        pltpu.async_copy(tab_hbm.at[idx_v], rows_v, sem).wait()

        def scale_row(r, carry2):
            for j in range(EMBED // LANES):
                sl = pl.ds(j * LANES, LANES)
                rows_v[r, sl] = rows_v[r, sl] * SCALE
            return carry2

        lax.fori_loop(0, CHUNK, scale_row, 0)
        pltpu.sync_copy(rows_v, out_hbm.at[pl.ds(start, CHUNK)])
        return carry

    lax.fori_loop(0, n_chunks, chunk_body, 0)


def kernel(x, table):
    b, h = x.shape
    n = b * h
    xf = x.reshape(n).astype(jnp.int32)

    mesh = plsc.VectorSubcoreMesh(core_axis_name="c", subcore_axis_name="s")
    k = functools.partial(
        pl.kernel,
        out_type=jax.ShapeDtypeStruct((n, EMBED), jnp.float32),
        mesh=mesh,
        scratch_types=[
            pltpu.VMEM((CHUNK,), jnp.int32),
            pltpu.VMEM((CHUNK, EMBED), jnp.float32),
            pltpu.SemaphoreType.DMA,
        ],
        compiler_params=pltpu.CompilerParams(use_tc_tiling_on_sc=False),
    )(_body)
    out = k(xf, table)
    return out.reshape(b, h, EMBED)


# speedup vs baseline: 1.1166x; 1.1166x over previous
import functools

import jax
import jax.numpy as jnp
from jax import lax
from jax.experimental import pallas as pl
from jax.experimental.pallas import tpu as pltpu
from jax.experimental.pallas import tpu_sc as plsc

EMBED = 64
LANES = 16
NUM_WORKERS = 32
CHUNK = 400
NB = 4
SCALE = 8.0


def _body(x_hbm, tab_hbm, out_hbm, idx_v, rows_v, gsem, ssem):
    wid = lax.axis_index("s") * 2 + lax.axis_index("c")
    n_total = x_hbm.shape[0]
    per_w = n_total // NUM_WORKERS
    n_chunks = per_w // CHUNK
    base = wid * per_w

    def fill(c, b):
        start = base + c * CHUNK
        pltpu.sync_copy(x_hbm.at[pl.ds(start, CHUNK)], idx_v.at[b])
        pltpu.async_copy(tab_hbm.at[idx_v.at[b]], rows_v.at[b], gsem.at[b])

    def wait_gather(b):
        pltpu.make_async_copy(
            tab_hbm.at[pl.ds(0, CHUNK)], rows_v.at[b], gsem.at[b]
        ).wait()

    def store(c, b):
        start = base + c * CHUNK
        pltpu.async_copy(
            rows_v.at[b], out_hbm.at[pl.ds(start, CHUNK)], ssem.at[b]
        )

    def wait_store(b):
        pltpu.make_async_copy(
            rows_v.at[b], out_hbm.at[pl.ds(base, CHUNK)], ssem.at[b]
        ).wait()

    def scale(b):
        @plsc.parallel_loop(0, CHUNK, step=1, unroll=8)
        def _(r):
            for j in range(EMBED // LANES):
                sl = pl.ds(j * LANES, LANES)
                rows_v[b, r, sl] = rows_v[b, r, sl] * SCALE

    fill(0, 0)
    fill(1, 1)

    def group(g, carry):
        for b in range(NB):
            c = g * NB + b
            br = (b + 2) % NB
            cr = c + 2

            @pl.when(cr < n_chunks)
            def _():
                @pl.when(c >= 2)
                def _():
                    wait_store(br)

                fill(cr, br)

            wait_gather(b)
            scale(b)
            store(c, b)
        return carry

    lax.fori_loop(0, n_chunks // NB, group, 0)
    for b in range(NB):
        wait_store(b)


def kernel(x, table):
    b, h = x.shape
    n = b * h
    xf = x.reshape(n).astype(jnp.int32)

    mesh = plsc.VectorSubcoreMesh(core_axis_name="c", subcore_axis_name="s")
    k = functools.partial(
        pl.kernel,
        out_type=jax.ShapeDtypeStruct((n, EMBED), jnp.float32),
        mesh=mesh,
        scratch_types=[
            pltpu.VMEM((NB, CHUNK), jnp.int32),
            pltpu.VMEM((NB, CHUNK, EMBED), jnp.float32),
            pltpu.SemaphoreType.DMA((NB,)),
            pltpu.SemaphoreType.DMA((NB,)),
        ],
        compiler_params=pltpu.CompilerParams(use_tc_tiling_on_sc=False),
    )(_body)
    out = k(xf, table)
    return out.reshape(b, h, EMBED)


# speedup vs baseline: 1.3637x; 1.2213x over previous
import functools

import jax
import jax.numpy as jnp
from jax import lax
from jax.experimental import pallas as pl
from jax.experimental.pallas import tpu as pltpu
from jax.experimental.pallas import tpu_sc as plsc

EMBED = 64
LANES = 16
NUM_WORKERS = 32
CHUNK = 200
PADDED = 128
NB = 4
SCALE = 8.0


def _body(x_hbm, tab_hbm, out_hbm, *scratch):
    idx = scratch[0:NB]
    rows = scratch[NB:2 * NB]
    gsem = scratch[2 * NB:3 * NB]
    ssem = scratch[3 * NB:4 * NB]

    wid = lax.axis_index("s") * 2 + lax.axis_index("c")
    n_total = x_hbm.shape[0]
    per_w = n_total // NUM_WORKERS
    n_chunks = per_w // CHUNK
    base = wid * per_w

    def fill(c, b):
        start = base + c * CHUNK
        pltpu.sync_copy(x_hbm.at[pl.ds(start, CHUNK)], idx[b])
        pltpu.async_copy(tab_hbm.at[idx[b]], rows[b], gsem[b])

    def wait_gather(b):
        pltpu.make_async_copy(
            tab_hbm.at[pl.ds(0, CHUNK)], rows[b], gsem[b]
        ).wait()

    def store(c, b):
        start = base + c * CHUNK
        pltpu.async_copy(rows[b], out_hbm.at[pl.ds(start, CHUNK)], ssem[b])

    def wait_store(b):
        pltpu.make_async_copy(
            rows[b], out_hbm.at[pl.ds(base, CHUNK)], ssem[b]
        ).wait()

    def scale(b):
        rb = rows[b]

        @plsc.parallel_loop(0, CHUNK, step=1, unroll=8)
        def _(r):
            for j in range(EMBED // LANES):
                sl = pl.ds(j * LANES, LANES)
                rb[r, sl] = rb[r, sl] * SCALE

    fill(0, 0)
    fill(1, 1)

    def group(g, carry):
        for b in range(NB):
            c = g * NB + b
            br = (b + 2) % NB
            cr = c + 2

            @pl.when(cr < n_chunks)
            def _():
                @pl.when(c >= 2)
                def _():
                    wait_store(br)

                fill(cr, br)

            wait_gather(b)
            scale(b)
            store(c, b)
        return carry

    lax.fori_loop(0, n_chunks // NB, group, 0)
    for b in range(NB):
        wait_store(b)


def kernel(x, table):
    b, h = x.shape
    n = b * h
    xf = x.reshape(n).astype(jnp.int32)
    tab128 = jnp.pad(table, ((0, 0), (0, PADDED - EMBED)))

    mesh = plsc.VectorSubcoreMesh(core_axis_name="c", subcore_axis_name="s")
    scratch = (
        [pltpu.VMEM((CHUNK,), jnp.int32) for _ in range(NB)]
        + [pltpu.VMEM((CHUNK, PADDED), jnp.float32) for _ in range(NB)]
        + [pltpu.SemaphoreType.DMA for _ in range(2 * NB)]
    )
    k = functools.partial(
        pl.kernel,
        out_type=jax.ShapeDtypeStruct((n, PADDED), jnp.float32),
        mesh=mesh,
        scratch_types=scratch,
        compiler_params=pltpu.CompilerParams(use_tc_tiling_on_sc=True),
    )(_body)
    out = k(xf, tab128)
    return out[:, :EMBED].reshape(b, h, EMBED)
